# speedup vs baseline: 1.0040x; 1.0040x over previous
import jax
import jax.numpy as jnp
from jax.experimental import pallas as pl

N = 10000
M = 5000
E = 320000
D = 128

TILE_E = 1000


def _ln(x, eps=1e-5):
    mu = jnp.mean(x, axis=-1, keepdims=True)
    var = jnp.var(x, axis=-1, keepdims=True)
    return (x - mu) / jnp.sqrt(var + eps)


def _mlp_e_kernel(x_ref, eg_ref, erw_ref, W1_ref, b1_ref, W2_ref, b2_ref,
                  beta_ref, out_ref):
    x = x_ref[...]
    x = _ln(x)
    h = x @ W1_ref[...] + b1_ref[...]
    h = jax.nn.relu(_ln(h))
    o = h @ W2_ref[...] + b2_ref[...]
    beta = beta_ref[0]
    out = beta * o + (1.0 - beta) * eg_ref[...]
    out_ref[...] = out * erw_ref[...]


def _mlp_e(x, eg, erw, W1, b1, W2, b2, beta):
    grid = (E // TILE_E,)
    return pl.pallas_call(
        _mlp_e_kernel,
        grid=grid,
        in_specs=[
            pl.BlockSpec((TILE_E, 2 * D), lambda i: (i, 0)),
            pl.BlockSpec((TILE_E, D), lambda i: (i, 0)),
            pl.BlockSpec((TILE_E, 1), lambda i: (i, 0)),
            pl.BlockSpec((2 * D, D), lambda i: (0, 0)),
            pl.BlockSpec((D,), lambda i: (0,)),
            pl.BlockSpec((D, D), lambda i: (0, 0)),
            pl.BlockSpec((D,), lambda i: (0,)),
            pl.BlockSpec((1,), lambda i: (0,)),
        ],
        out_specs=pl.BlockSpec((TILE_E, D), lambda i: (i, 0)),
        out_shape=jax.ShapeDtypeStruct((E, D), jnp.float32),
    )(x, eg, erw, W1, b1, W2, b2, jnp.full((1,), beta, jnp.float32))


def _mlp(x, W1, b1, W2, b2):
    x = _ln(x)
    h = jax.nn.relu(_ln(x @ W1 + b1))
    return h @ W2 + b2


def _scatter_mean(vals, idx, num):
    s = jax.ops.segment_sum(vals, idx, num_segments=num)
    c = jax.ops.segment_sum(jnp.ones((vals.shape[0], 1), vals.dtype), idx,
                            num_segments=num)
    return s / jnp.maximum(c, 1.0)


def kernel(v, e, v0, e0, n_reg_weight, e_reg_weight, n_reg_sum, e_reg_sum,
           W1n, b1n, W2n, b2n, W1e, b1e, W2e, b2e, W1a, b1a, W2a, b2a,
           vidx, eidx, alpha, beta):
    node_msg = _mlp(v, W1n, b1n, W2n, b2n)
    node_msg = node_msg[vidx] * n_reg_weight
    edge = _scatter_mean(node_msg, eidx, M)
    edge = (1.0 - alpha) * edge + alpha * e0
    edge = edge / e_reg_sum

    x = jnp.concatenate((v[vidx], edge[eidx]), axis=-1)
    edge_msg = _mlp_e(x, e[eidx], e_reg_weight, W1e, b1e, W2e, b2e, beta)

    node = _scatter_mean(edge_msg, vidx, N)
    node = node / n_reg_sum
    node = (1.0 - alpha) * node + alpha * v0
    node = beta * _mlp(node, W1a, b1a, W2a, b2a) + (1.0 - beta) * node
    return (node, edge)
